# baseline (speedup 1.0000x reference)
import functools

import jax
import jax.numpy as jnp
from jax import lax
from jax.experimental import pallas as pl
from jax.experimental.pallas import tpu as pltpu
from jax.experimental.pallas import tpu_sc as plsc

N = 10000
E = 320000
D = 128

NC = 2
NS = 16
NW = NC * NS
N_PAD = 10112
ROWS_PER_TILE = N_PAD // NS
K = 128
CHUNKS = 80
E_PAD = NW * CHUNKS * K
PAIRS = CHUNKS // 2
B = 632
GRID = N_PAD // B

_mesh = plsc.VectorSubcoreMesh(core_axis_name="c", subcore_axis_name="s")


def _fill(ref, rows, value):
  v = jnp.full((16,), value, jnp.float32)

  def body(r, _):
    for j in range(8):
      ref[r, pl.ds(j * 16, 16)] = v
    return 0

  lax.fori_loop(0, rows, body, 0)


def _zero_shared_slice(zer_v, shared, s):
  full, rem = divmod(ROWS_PER_TILE, 128)
  for k in range(full):
    pltpu.sync_copy(zer_v, shared.at[pl.ds(s * ROWS_PER_TILE + k * 128, 128)])
  if rem:
    pltpu.sync_copy(
        zer_v.at[pl.ds(0, rem)],
        shared.at[pl.ds(s * ROWS_PER_TILE + full * 128, rem)])


@functools.partial(
    pl.kernel,
    out_type=jax.ShapeDtypeStruct((NC, N_PAD, D), jnp.float32),
    mesh=_mesh,
    scratch_types=[
        pltpu.VMEM((K,), jnp.int32),
        pltpu.VMEM((K,), jnp.int32),
        pltpu.VMEM((K, D), jnp.float32),
        pltpu.VMEM((128, D), jnp.float32),
        pltpu.VMEM_SHARED((N_PAD, D), jnp.float32),
        pltpu.SemaphoreType.DMA,
        pltpu.SemaphoreType.DMA,
    ],
)
def _deg_kernel(dst_hbm, out_hbm, dst_a, dst_b, ones_v, zer_v, hist,
                semA, semB):
  c = lax.axis_index("c")
  s = lax.axis_index("s")
  wid = c * NS + s

  _fill(ones_v, K, 1.0)
  _fill(zer_v, 128, 0.0)
  _zero_shared_slice(zer_v, hist, s)
  plsc.subcore_barrier()

  pltpu.sync_copy(dst_hbm.at[wid, 0], dst_a)
  pltpu.async_copy(ones_v, hist.at[dst_a], semA, add=True)
  pltpu.sync_copy(dst_hbm.at[wid, 1], dst_b)
  pltpu.async_copy(ones_v, hist.at[dst_b], semB, add=True)

  def body(j, _):
    a = 2 * j
    pltpu.make_async_copy(ones_v, hist.at[dst_a], semA).wait()
    pltpu.sync_copy(dst_hbm.at[wid, a], dst_a)
    pltpu.async_copy(ones_v, hist.at[dst_a], semA, add=True)
    pltpu.make_async_copy(ones_v, hist.at[dst_b], semB).wait()
    pltpu.sync_copy(dst_hbm.at[wid, a + 1], dst_b)
    pltpu.async_copy(ones_v, hist.at[dst_b], semB, add=True)
    return 0

  lax.fori_loop(1, PAIRS, body, 0)
  pltpu.make_async_copy(ones_v, hist.at[dst_a], semA).wait()
  pltpu.make_async_copy(ones_v, hist.at[dst_b], semB).wait()
  plsc.subcore_barrier()

  pltpu.sync_copy(
      hist.at[pl.ds(s * ROWS_PER_TILE, ROWS_PER_TILE)],
      out_hbm.at[c, pl.ds(s * ROWS_PER_TILE, ROWS_PER_TILE)],
  )


@functools.partial(
    pl.kernel,
    out_type=jax.ShapeDtypeStruct((NC, N_PAD, D), jnp.float32),
    mesh=_mesh,
    scratch_types=[
        pltpu.VMEM((K,), jnp.int32),
        pltpu.VMEM((K,), jnp.int32),
        pltpu.VMEM((K,), jnp.int32),
        pltpu.VMEM((K,), jnp.int32),
        pltpu.VMEM((K, D), jnp.float32),
        pltpu.VMEM((K, D), jnp.float32),
        pltpu.VMEM((128, D), jnp.float32),
        pltpu.VMEM_SHARED((N_PAD, D), jnp.float32),
        pltpu.SemaphoreType.DMA,
        pltpu.SemaphoreType.DMA,
        pltpu.SemaphoreType.DMA,
        pltpu.SemaphoreType.DMA,
    ],
)
def _edge_kernel(y_hbm, src_hbm, dst_hbm, out_hbm, src_a, dst_a, src_b, dst_b,
                 rows_a, rows_b, zer_v, acc, gA, gB, sA, sB):
  c = lax.axis_index("c")
  s = lax.axis_index("s")
  wid = c * NS + s

  _fill(zer_v, 128, 0.0)
  _zero_shared_slice(zer_v, acc, s)
  plsc.subcore_barrier()

  def body(j, _):
    a = 2 * j
    b = a + 1

    @pl.when(j > 0)
    def _():
      pltpu.make_async_copy(rows_a, acc.at[dst_a], sA).wait()

    pltpu.sync_copy(src_hbm.at[wid, a], src_a)
    pltpu.sync_copy(dst_hbm.at[wid, a], dst_a)
    pltpu.async_copy(y_hbm.at[src_a], rows_a, gA)

    @pl.when(j > 0)
    def _():
      pltpu.make_async_copy(rows_b, acc.at[dst_b], sB).wait()

    pltpu.sync_copy(src_hbm.at[wid, b], src_b)
    pltpu.sync_copy(dst_hbm.at[wid, b], dst_b)
    pltpu.async_copy(y_hbm.at[src_b], rows_b, gB)

    pltpu.make_async_copy(y_hbm.at[src_a], rows_a, gA).wait()
    pltpu.async_copy(rows_a, acc.at[dst_a], sA, add=True)
    pltpu.make_async_copy(y_hbm.at[src_b], rows_b, gB).wait()
    pltpu.async_copy(rows_b, acc.at[dst_b], sB, add=True)
    return 0

  lax.fori_loop(0, PAIRS, body, 0)
  pltpu.make_async_copy(rows_a, acc.at[dst_a], sA).wait()
  pltpu.make_async_copy(rows_b, acc.at[dst_b], sB).wait()
  plsc.subcore_barrier()

  pltpu.sync_copy(
      acc.at[pl.ds(s * ROWS_PER_TILE, ROWS_PER_TILE)],
      out_hbm.at[c, pl.ds(s * ROWS_PER_TILE, ROWS_PER_TILE)],
  )


def _tc1_body(hist_ref, x_ref, w_ref, y_ref, dinv_ref):
  h = hist_ref[...]
  deg = h[0, :, :] + h[1, :, :]
  i = pl.program_id(0)
  row = lax.broadcasted_iota(jnp.int32, (B, D), 0) + i * B
  deg = deg + jnp.where(row < N, 1.0, 0.0)
  d128 = jnp.where(deg > 0, lax.rsqrt(jnp.maximum(deg, 1e-12)), 0.0)
  dinv_ref[...] = d128
  xw = jnp.dot(x_ref[...], w_ref[...], preferred_element_type=jnp.float32)
  y_ref[...] = xw * d128


def _tc2_body(part_ref, y_ref, dinv_ref, b_ref, w_ref, out_ref):
  p = part_ref[0, :, :] + part_ref[1, :, :]
  dinv = dinv_ref[...]
  h = jnp.maximum((p + y_ref[...]) * dinv + b_ref[...], 0.0)
  out_ref[...] = (
      jnp.dot(h, w_ref[...], preferred_element_type=jnp.float32) * dinv)


def _tc3_body(part_ref, y_ref, dinv_ref, b_ref, out_ref):
  p = part_ref[0, :, :] + part_ref[1, :, :]
  out_ref[...] = (p + y_ref[...]) * dinv_ref[...] + b_ref[...]


_row_spec = pl.BlockSpec((B, D), lambda i: (i, 0))
_part_spec = pl.BlockSpec((NC, B, D), lambda i: (0, i, 0))
_mat_spec = pl.BlockSpec((D, D), lambda i: (0, 0))
_bias_spec = pl.BlockSpec((1, D), lambda i: (0, 0))

_tc1 = pl.pallas_call(
    _tc1_body,
    grid=(GRID,),
    in_specs=[_part_spec, _row_spec, _mat_spec],
    out_specs=[_row_spec, _row_spec],
    out_shape=[
        jax.ShapeDtypeStruct((N_PAD, D), jnp.float32),
        jax.ShapeDtypeStruct((N_PAD, D), jnp.float32),
    ],
)

_tc2 = pl.pallas_call(
    _tc2_body,
    grid=(GRID,),
    in_specs=[_part_spec, _row_spec, _row_spec, _bias_spec, _mat_spec],
    out_specs=_row_spec,
    out_shape=jax.ShapeDtypeStruct((N_PAD, D), jnp.float32),
)

_tc3 = pl.pallas_call(
    _tc3_body,
    grid=(GRID,),
    in_specs=[_part_spec, _row_spec, _row_spec, _bias_spec],
    out_specs=_row_spec,
    out_shape=jax.ShapeDtypeStruct((N_PAD, D), jnp.float32),
)


@jax.jit
def kernel(x, edge_index, W1, b1, W2, b2):
  pad = jnp.full((E_PAD - E,), N, jnp.int32)
  src = jnp.concatenate([edge_index[0].astype(jnp.int32), pad])
  dst = jnp.concatenate([edge_index[1].astype(jnp.int32), pad])
  src = src.reshape(NW, CHUNKS, K)
  dst = dst.reshape(NW, CHUNKS, K)
  b1r = b1.reshape(1, D)
  b2r = b2.reshape(1, D)

  x_pad = jnp.zeros((N_PAD, D), jnp.float32).at[:N].set(x)
  hist = _deg_kernel(dst)
  y1, dinv = _tc1(hist, x_pad, W1)
  part1 = _edge_kernel(y1, src, dst)
  y2 = _tc2(part1, y1, dinv, b1r, W2)
  part2 = _edge_kernel(y2, src, dst)
  out = _tc3(part2, y2, dinv, b2r)
  return out[:N]


# speedup vs baseline: 2.7278x; 2.7278x over previous
import functools

import jax
import jax.numpy as jnp
from jax import lax
from jax.experimental import pallas as pl
from jax.experimental.pallas import tpu as pltpu
from jax.experimental.pallas import tpu_sc as plsc

N = 10000
E = 320000
D = 128

NC = 2
NS = 16
NW = NC * NS
N_PAD = 10112
ROWS_PER_TILE = N_PAD // NS
K = 125
CHUNKS = 80
PAIRS = CHUNKS // 2
B = 632
GRID = N_PAD // B

_mesh = plsc.VectorSubcoreMesh(core_axis_name="c", subcore_axis_name="s")


def _fill(ref, rows, value):
  v = jnp.full((16,), value, jnp.float32)

  def body(r, _):
    for j in range(8):
      ref[r, pl.ds(j * 16, 16)] = v
    return 0

  lax.fori_loop(0, rows, body, 0)


def _zero_shared_slice(zer_v, shared, s):
  full, rem = divmod(ROWS_PER_TILE, 128)
  for k in range(full):
    pltpu.sync_copy(zer_v, shared.at[pl.ds(s * ROWS_PER_TILE + k * 128, 128)])
  if rem:
    pltpu.sync_copy(
        zer_v.at[pl.ds(0, rem)],
        shared.at[pl.ds(s * ROWS_PER_TILE + full * 128, rem)])


@functools.partial(
    pl.kernel,
    out_type=jax.ShapeDtypeStruct((NC, N_PAD, D), jnp.float32),
    mesh=_mesh,
    scratch_types=[
        pltpu.VMEM((K,), jnp.int32),
        pltpu.VMEM((K,), jnp.int32),
        pltpu.VMEM((K, D), jnp.float32),
        pltpu.VMEM((128, D), jnp.float32),
        pltpu.VMEM_SHARED((N_PAD, D), jnp.float32),
        pltpu.SemaphoreType.DMA,
        pltpu.SemaphoreType.DMA,
    ],
)
def _deg_kernel(dst_hbm, out_hbm, dst_a, dst_b, ones_v, zer_v, hist,
                semA, semB):
  c = lax.axis_index("c")
  s = lax.axis_index("s")
  wid = c * NS + s

  _fill(ones_v, K, 1.0)
  _fill(zer_v, 128, 0.0)
  _zero_shared_slice(zer_v, hist, s)
  plsc.subcore_barrier()

  pltpu.sync_copy(dst_hbm.at[wid, 0], dst_a)
  pltpu.async_copy(ones_v, hist.at[dst_a], semA, add=True)
  pltpu.sync_copy(dst_hbm.at[wid, 1], dst_b)
  pltpu.async_copy(ones_v, hist.at[dst_b], semB, add=True)

  def body(j, _):
    a = 2 * j
    pltpu.make_async_copy(ones_v, hist.at[dst_a], semA).wait()
    pltpu.sync_copy(dst_hbm.at[wid, a], dst_a)
    pltpu.async_copy(ones_v, hist.at[dst_a], semA, add=True)
    pltpu.make_async_copy(ones_v, hist.at[dst_b], semB).wait()
    pltpu.sync_copy(dst_hbm.at[wid, a + 1], dst_b)
    pltpu.async_copy(ones_v, hist.at[dst_b], semB, add=True)
    return 0

  lax.fori_loop(1, PAIRS, body, 0)
  pltpu.make_async_copy(ones_v, hist.at[dst_a], semA).wait()
  pltpu.make_async_copy(ones_v, hist.at[dst_b], semB).wait()
  plsc.subcore_barrier()

  pltpu.sync_copy(
      hist.at[pl.ds(s * ROWS_PER_TILE, ROWS_PER_TILE)],
      out_hbm.at[c, pl.ds(s * ROWS_PER_TILE, ROWS_PER_TILE)],
  )


@functools.partial(
    pl.kernel,
    out_type=jax.ShapeDtypeStruct((NC, N_PAD, D), jnp.float32),
    mesh=_mesh,
    scratch_types=[
        pltpu.VMEM((K,), jnp.int32),
        pltpu.VMEM((K,), jnp.int32),
        pltpu.VMEM((K,), jnp.int32),
        pltpu.VMEM((K,), jnp.int32),
        pltpu.VMEM((K, D), jnp.float32),
        pltpu.VMEM((K, D), jnp.float32),
        pltpu.VMEM((128, D), jnp.float32),
        pltpu.VMEM_SHARED((N_PAD, D), jnp.float32),
        pltpu.SemaphoreType.DMA,
        pltpu.SemaphoreType.DMA,
        pltpu.SemaphoreType.DMA,
        pltpu.SemaphoreType.DMA,
    ],
)
def _edge_kernel(y_hbm, src_hbm, dst_hbm, out_hbm, src_a, dst_a, src_b, dst_b,
                 rows_a, rows_b, zer_v, acc, gA, gB, sA, sB):
  c = lax.axis_index("c")
  s = lax.axis_index("s")
  wid = c * NS + s

  _fill(zer_v, 128, 0.0)
  _zero_shared_slice(zer_v, acc, s)
  plsc.subcore_barrier()

  def body(j, _):
    a = 2 * j
    b = a + 1

    @pl.when(j > 0)
    def _():
      pltpu.make_async_copy(rows_a, acc.at[dst_a], sA).wait()

    pltpu.sync_copy(src_hbm.at[wid, a], src_a)
    pltpu.sync_copy(dst_hbm.at[wid, a], dst_a)
    pltpu.async_copy(y_hbm.at[src_a], rows_a, gA)

    @pl.when(j > 0)
    def _():
      pltpu.make_async_copy(rows_b, acc.at[dst_b], sB).wait()

    pltpu.sync_copy(src_hbm.at[wid, b], src_b)
    pltpu.sync_copy(dst_hbm.at[wid, b], dst_b)
    pltpu.async_copy(y_hbm.at[src_b], rows_b, gB)

    pltpu.make_async_copy(y_hbm.at[src_a], rows_a, gA).wait()
    pltpu.async_copy(rows_a, acc.at[dst_a], sA, add=True)
    pltpu.make_async_copy(y_hbm.at[src_b], rows_b, gB).wait()
    pltpu.async_copy(rows_b, acc.at[dst_b], sB, add=True)
    return 0

  lax.fori_loop(0, PAIRS, body, 0)
  pltpu.make_async_copy(rows_a, acc.at[dst_a], sA).wait()
  pltpu.make_async_copy(rows_b, acc.at[dst_b], sB).wait()
  plsc.subcore_barrier()

  pltpu.sync_copy(
      acc.at[pl.ds(s * ROWS_PER_TILE, ROWS_PER_TILE)],
      out_hbm.at[c, pl.ds(s * ROWS_PER_TILE, ROWS_PER_TILE)],
  )


def _tc0_body(x_ref, w_ref, xw_ref):
  xw_ref[...] = jnp.dot(x_ref[...], w_ref[...],
                        preferred_element_type=jnp.float32)


def _tc1_body(hist_ref, xw_ref, y_ref, dinv_ref):
  h = hist_ref[...]
  deg = h[0, :, :] + h[1, :, :]
  i = pl.program_id(0)
  row = lax.broadcasted_iota(jnp.int32, (B, D), 0) + i * B
  deg = deg + jnp.where(row < N, 1.0, 0.0)
  d128 = jnp.where(deg > 0, lax.rsqrt(jnp.maximum(deg, 1e-12)), 0.0)
  dinv_ref[...] = d128
  y_ref[...] = xw_ref[...] * d128


def _tc2_body(part_ref, y_ref, dinv_ref, b_ref, w_ref, out_ref):
  p = part_ref[0, :, :] + part_ref[1, :, :]
  dinv = dinv_ref[...]
  h = jnp.maximum((p + y_ref[...]) * dinv + b_ref[...], 0.0)
  out_ref[...] = (
      jnp.dot(h, w_ref[...], preferred_element_type=jnp.float32) * dinv)


def _tc3_body(part_ref, y_ref, dinv_ref, b_ref, out_ref):
  p = part_ref[0, :, :] + part_ref[1, :, :]
  out_ref[...] = (p + y_ref[...]) * dinv_ref[...] + b_ref[...]


_row_spec = pl.BlockSpec((B, D), lambda i: (i, 0))
_part_spec = pl.BlockSpec((NC, B, D), lambda i: (0, i, 0))
_mat_spec = pl.BlockSpec((D, D), lambda i: (0, 0))
_bias_spec = pl.BlockSpec((1, D), lambda i: (0, 0))

_tc0 = pl.pallas_call(
    _tc0_body,
    grid=(GRID,),
    in_specs=[_row_spec, _mat_spec],
    out_specs=_row_spec,
    out_shape=jax.ShapeDtypeStruct((N_PAD, D), jnp.float32),
)

_tc1 = pl.pallas_call(
    _tc1_body,
    grid=(GRID,),
    in_specs=[_part_spec, _row_spec],
    out_specs=[_row_spec, _row_spec],
    out_shape=[
        jax.ShapeDtypeStruct((N_PAD, D), jnp.float32),
        jax.ShapeDtypeStruct((N_PAD, D), jnp.float32),
    ],
)

_tc2 = pl.pallas_call(
    _tc2_body,
    grid=(GRID,),
    in_specs=[_part_spec, _row_spec, _row_spec, _bias_spec, _mat_spec],
    out_specs=_row_spec,
    out_shape=jax.ShapeDtypeStruct((N_PAD, D), jnp.float32),
)

_tc3 = pl.pallas_call(
    _tc3_body,
    grid=(GRID,),
    in_specs=[_part_spec, _row_spec, _row_spec, _bias_spec],
    out_specs=_row_spec,
    out_shape=jax.ShapeDtypeStruct((N_PAD, D), jnp.float32),
)


@jax.jit
def kernel(x, edge_index, W1, b1, W2, b2):
  src = edge_index[0].astype(jnp.int32).reshape(NW, CHUNKS, K)
  dst = edge_index[1].astype(jnp.int32).reshape(NW, CHUNKS, K)
  b1r = b1.reshape(1, D)
  b2r = b2.reshape(1, D)

  x_pad = jnp.zeros((N_PAD, D), jnp.float32).at[:N].set(x)
  xw1 = _tc0(x_pad, W1)
  hist = _deg_kernel(dst)
  y1, dinv = _tc1(hist, xw1)
  part1 = _edge_kernel(y1, src, dst)
  y2 = _tc2(part1, y1, dinv, b1r, W2)
  part2 = _edge_kernel(y2, src, dst)
  out = _tc3(part2, y2, dinv, b2r)
  return out[:N]


# speedup vs baseline: 2.7430x; 1.0056x over previous
import functools

import jax
import jax.numpy as jnp
from jax import lax
from jax.experimental import pallas as pl
from jax.experimental.pallas import tpu as pltpu
from jax.experimental.pallas import tpu_sc as plsc

N = 10000
E = 320000
D = 128

NC = 2
NS = 16
NW = NC * NS
N_PAD = 10112
ROWS_PER_TILE = N_PAD // NS
K = 125
CHUNKS = 80
PAIRS = CHUNKS // 2
B = 632
GRID = N_PAD // B

_mesh = plsc.VectorSubcoreMesh(core_axis_name="c", subcore_axis_name="s")


def _fill(ref, rows, value):
  v = jnp.full((16,), value, jnp.float32)

  def body(r, _):
    for j in range(8):
      ref[r, pl.ds(j * 16, 16)] = v
    return 0

  lax.fori_loop(0, rows, body, 0)


def _zero_shared_slice(zer_v, shared, s):
  full, rem = divmod(ROWS_PER_TILE, 128)
  for k in range(full):
    pltpu.sync_copy(zer_v, shared.at[pl.ds(s * ROWS_PER_TILE + k * 128, 128)])
  if rem:
    pltpu.sync_copy(
        zer_v.at[pl.ds(0, rem)],
        shared.at[pl.ds(s * ROWS_PER_TILE + full * 128, rem)])


@functools.partial(
    pl.kernel,
    out_type=jax.ShapeDtypeStruct((NC, N_PAD, D), jnp.float32),
    mesh=_mesh,
    scratch_types=[
        pltpu.VMEM((K,), jnp.int32),
        pltpu.VMEM((K,), jnp.int32),
        pltpu.VMEM((K, D), jnp.float32),
        pltpu.VMEM((128, D), jnp.float32),
        pltpu.VMEM_SHARED((N_PAD, D), jnp.float32),
        pltpu.SemaphoreType.DMA,
        pltpu.SemaphoreType.DMA,
    ],
)
def _deg_kernel(dst_hbm, out_hbm, dst_a, dst_b, ones_v, zer_v, hist,
                semA, semB):
  c = lax.axis_index("c")
  s = lax.axis_index("s")
  wid = c * NS + s

  _fill(ones_v, K, 1.0)
  _fill(zer_v, 128, 0.0)
  _zero_shared_slice(zer_v, hist, s)
  plsc.subcore_barrier()

  pltpu.sync_copy(dst_hbm.at[wid, 0], dst_a)
  pltpu.async_copy(ones_v, hist.at[dst_a], semA, add=True)
  pltpu.sync_copy(dst_hbm.at[wid, 1], dst_b)
  pltpu.async_copy(ones_v, hist.at[dst_b], semB, add=True)

  def body(j, _):
    a = 2 * j
    pltpu.make_async_copy(ones_v, hist.at[dst_a], semA).wait()
    pltpu.sync_copy(dst_hbm.at[wid, a], dst_a)
    pltpu.async_copy(ones_v, hist.at[dst_a], semA, add=True)
    pltpu.make_async_copy(ones_v, hist.at[dst_b], semB).wait()
    pltpu.sync_copy(dst_hbm.at[wid, a + 1], dst_b)
    pltpu.async_copy(ones_v, hist.at[dst_b], semB, add=True)
    return 0

  lax.fori_loop(1, PAIRS, body, 0)
  pltpu.make_async_copy(ones_v, hist.at[dst_a], semA).wait()
  pltpu.make_async_copy(ones_v, hist.at[dst_b], semB).wait()
  plsc.subcore_barrier()

  pltpu.sync_copy(
      hist.at[pl.ds(s * ROWS_PER_TILE, ROWS_PER_TILE)],
      out_hbm.at[c, pl.ds(s * ROWS_PER_TILE, ROWS_PER_TILE)],
  )


@functools.partial(
    pl.kernel,
    out_type=jax.ShapeDtypeStruct((NC, N_PAD, D), jnp.float32),
    mesh=_mesh,
    scratch_types=[
        pltpu.VMEM((K,), jnp.int32),
        pltpu.VMEM((K,), jnp.int32),
        pltpu.VMEM((K,), jnp.int32),
        pltpu.VMEM((K,), jnp.int32),
        pltpu.VMEM((K, D), jnp.float32),
        pltpu.VMEM((K, D), jnp.float32),
        pltpu.VMEM((128, D), jnp.float32),
        pltpu.VMEM_SHARED((N_PAD, D), jnp.float32),
        pltpu.SemaphoreType.DMA,
        pltpu.SemaphoreType.DMA,
        pltpu.SemaphoreType.DMA,
        pltpu.SemaphoreType.DMA,
    ],
)
def _edge_kernel(y_hbm, src_hbm, dst_hbm, out_hbm, src_a, dst_a, src_b, dst_b,
                 rows_a, rows_b, zer_v, acc, gA, gB, sA, sB):
  c = lax.axis_index("c")
  s = lax.axis_index("s")
  wid = c * NS + s

  _fill(zer_v, 128, 0.0)
  _zero_shared_slice(zer_v, acc, s)
  plsc.subcore_barrier()

  pltpu.sync_copy(src_hbm.at[wid, 0], src_a)
  pltpu.sync_copy(dst_hbm.at[wid, 0], dst_a)
  pltpu.async_copy(y_hbm.at[src_a], rows_a, gA)
  _fill(rows_b, K, 0.0)
  pltpu.sync_copy(dst_hbm.at[wid, 0], dst_b)
  pltpu.async_copy(rows_b, acc.at[dst_b], sB, add=True)

  def body(j, _):
    a = 2 * j
    b = a + 1
    pltpu.make_async_copy(rows_b, acc.at[dst_b], sB).wait()
    pltpu.sync_copy(src_hbm.at[wid, b], src_b)
    pltpu.sync_copy(dst_hbm.at[wid, b], dst_b)
    pltpu.make_async_copy(y_hbm.at[src_a], rows_a, gA).wait()
    pltpu.async_copy(rows_a, acc.at[dst_a], sA, add=True)
    pltpu.async_copy(y_hbm.at[src_b], rows_b, gB)
    pltpu.make_async_copy(rows_a, acc.at[dst_a], sA).wait()

    @pl.when(j < PAIRS - 1)
    def _():
      pltpu.sync_copy(src_hbm.at[wid, a + 2], src_a)
      pltpu.sync_copy(dst_hbm.at[wid, a + 2], dst_a)

    pltpu.make_async_copy(y_hbm.at[src_b], rows_b, gB).wait()
    pltpu.async_copy(rows_b, acc.at[dst_b], sB, add=True)

    @pl.when(j < PAIRS - 1)
    def _():
      pltpu.async_copy(y_hbm.at[src_a], rows_a, gA)

    return 0

  lax.fori_loop(0, PAIRS, body, 0)
  pltpu.make_async_copy(rows_b, acc.at[dst_b], sB).wait()
  plsc.subcore_barrier()

  pltpu.sync_copy(
      acc.at[pl.ds(s * ROWS_PER_TILE, ROWS_PER_TILE)],
      out_hbm.at[c, pl.ds(s * ROWS_PER_TILE, ROWS_PER_TILE)],
  )


def _tc0_body(x_ref, w_ref, xw_ref):
  xw_ref[...] = jnp.dot(x_ref[...], w_ref[...],
                        preferred_element_type=jnp.float32)


def _tc1_body(hist_ref, xw_ref, y_ref, dinv_ref):
  h = hist_ref[...]
  deg = h[0, :, :] + h[1, :, :]
  i = pl.program_id(0)
  row = lax.broadcasted_iota(jnp.int32, (B, D), 0) + i * B
  deg = deg + jnp.where(row < N, 1.0, 0.0)
  d128 = jnp.where(deg > 0, lax.rsqrt(jnp.maximum(deg, 1e-12)), 0.0)
  dinv_ref[...] = d128
  y_ref[...] = xw_ref[...] * d128


def _tc2_body(part_ref, y_ref, dinv_ref, b_ref, w_ref, out_ref):
  p = part_ref[0, :, :] + part_ref[1, :, :]
  dinv = dinv_ref[...]
  h = jnp.maximum((p + y_ref[...]) * dinv + b_ref[...], 0.0)
  out_ref[...] = (
      jnp.dot(h, w_ref[...], preferred_element_type=jnp.float32) * dinv)


def _tc3_body(part_ref, y_ref, dinv_ref, b_ref, out_ref):
  p = part_ref[0, :, :] + part_ref[1, :, :]
  out_ref[...] = (p + y_ref[...]) * dinv_ref[...] + b_ref[...]


_row_spec = pl.BlockSpec((B, D), lambda i: (i, 0))
_part_spec = pl.BlockSpec((NC, B, D), lambda i: (0, i, 0))
_mat_spec = pl.BlockSpec((D, D), lambda i: (0, 0))
_bias_spec = pl.BlockSpec((1, D), lambda i: (0, 0))

_tc0 = pl.pallas_call(
    _tc0_body,
    grid=(GRID,),
    in_specs=[_row_spec, _mat_spec],
    out_specs=_row_spec,
    out_shape=jax.ShapeDtypeStruct((N_PAD, D), jnp.float32),
)

_tc1 = pl.pallas_call(
    _tc1_body,
    grid=(GRID,),
    in_specs=[_part_spec, _row_spec],
    out_specs=[_row_spec, _row_spec],
    out_shape=[
        jax.ShapeDtypeStruct((N_PAD, D), jnp.float32),
        jax.ShapeDtypeStruct((N_PAD, D), jnp.float32),
    ],
)

_tc2 = pl.pallas_call(
    _tc2_body,
    grid=(GRID,),
    in_specs=[_part_spec, _row_spec, _row_spec, _bias_spec, _mat_spec],
    out_specs=_row_spec,
    out_shape=jax.ShapeDtypeStruct((N_PAD, D), jnp.float32),
)

_tc3 = pl.pallas_call(
    _tc3_body,
    grid=(GRID,),
    in_specs=[_part_spec, _row_spec, _row_spec, _bias_spec],
    out_specs=_row_spec,
    out_shape=jax.ShapeDtypeStruct((N_PAD, D), jnp.float32),
)


@jax.jit
def kernel(x, edge_index, W1, b1, W2, b2):
  src = edge_index[0].astype(jnp.int32).reshape(NW, CHUNKS, K)
  dst = edge_index[1].astype(jnp.int32).reshape(NW, CHUNKS, K)
  b1r = b1.reshape(1, D)
  b2r = b2.reshape(1, D)

  x_pad = jnp.zeros((N_PAD, D), jnp.float32).at[:N].set(x)
  xw1 = _tc0(x_pad, W1)
  hist = _deg_kernel(dst)
  y1, dinv = _tc1(hist, xw1)
  part1 = _edge_kernel(y1, src, dst)
  y2 = _tc2(part1, y1, dinv, b1r, W2)
  part2 = _edge_kernel(y2, src, dst)
  out = _tc3(part2, y2, dinv, b2r)
  return out[:N]
